# baseline (speedup 1.0000x reference)
import jax
import jax.numpy as jnp
from jax import lax
from jax.experimental import pallas as pl
from jax.experimental.pallas import tpu as pltpu
from jax.experimental.pallas import tpu_sc as plsc

f32 = jnp.float32

N = 10000
E = 320000
IN_D = 128
HID = 256
HALF = 128
OUT_D = 128

NSUB = 16
EB = 128
BLOCKS = 160
E_PAD = NSUB * BLOCKS * EB
CHK = 8
NCH = BLOCKS // CHK
DBLOCKS = 80
DNCH = DBLOCKS // CHK
ZROWS = 624
ACC_ROWS = 10008

_sc_mesh = plsc.VectorSubcoreMesh(core_axis_name="c", subcore_axis_name="s")


def _zero_slices(sid, dst_ref, z128):
  pltpu.sync_copy(z128, dst_ref.at[pl.ds(sid * ZROWS, ZROWS)])
  rem = ACC_ROWS - NSUB * ZROWS

  @pl.when(sid == 0)
  def _():
    pltpu.sync_copy(z128.at[pl.ds(0, rem)],
                    dst_ref.at[pl.ds(NSUB * ZROWS, rem)])


def _out_slices(sid, src_ref, out_ref):
  lo = sid * ZROWS
  pltpu.sync_copy(src_ref.at[pl.ds(lo, ZROWS)], out_ref.at[pl.ds(lo, ZROWS)])
  rem = ACC_ROWS - NSUB * ZROWS

  @pl.when(sid == 0)
  def _():
    pltpu.sync_copy(src_ref.at[pl.ds(NSUB * ZROWS, rem)],
                    out_ref.at[pl.ds(NSUB * ZROWS, rem)])


def _agg_body(h0, h1, srcr, dstr, z128, out0, out1, srcv, dstv, rows0, rows1,
              acc, sem, sem1):
  cid = lax.axis_index("c")
  sid = lax.axis_index("s")

  _zero_slices(sid, acc, z128)
  plsc.subcore_barrier()

  def run(h):
    def chunk(c, carry):
      pltpu.sync_copy(srcr.at[sid, pl.ds(c * CHK, CHK)], srcv)
      pltpu.sync_copy(dstr.at[sid, pl.ds(c * CHK, CHK)], dstv)
      pltpu.async_copy(h.at[srcv.at[0]], rows0, sem)
      pltpu.async_copy(h.at[srcv.at[1]], rows1, sem1)
      for k in range(CHK):
        cur = rows0 if k % 2 == 0 else rows1
        sm = sem if k % 2 == 0 else sem1
        pltpu.make_async_copy(h.at[srcv.at[k]], cur, sm).wait()
        pltpu.sync_copy(cur, acc.at[dstv.at[k]], add=True)
        if k + 2 < CHK:
          pltpu.async_copy(h.at[srcv.at[k + 2]], cur, sm)
      return carry
    lax.fori_loop(0, NCH, chunk, 0)

  @pl.when(cid == 0)
  def _():
    run(h0)

  @pl.when(cid == 1)
  def _():
    run(h1)

  plsc.subcore_barrier()

  @pl.when(cid == 0)
  def _():
    _out_slices(sid, acc, out0)

  @pl.when(cid == 1)
  def _():
    _out_slices(sid, acc, out1)


_agg = pl.kernel(
    _agg_body, mesh=_sc_mesh,
    out_type=[jax.ShapeDtypeStruct((ACC_ROWS, HALF), f32),
              jax.ShapeDtypeStruct((ACC_ROWS, HALF), f32)],
    scratch_types=[
        pltpu.VMEM((CHK, EB), jnp.int32),
        pltpu.VMEM((CHK, EB), jnp.int32),
        pltpu.VMEM((EB, HALF), f32),
        pltpu.VMEM((EB, HALF), f32),
        pltpu.VMEM_SHARED((ACC_ROWS, HALF), f32),
        pltpu.SemaphoreType.DMA,
        pltpu.SemaphoreType.DMA,
    ])


def _deg_body(dstr2, ones_h, z128, d0, d1, dstv, onesv, dacc):
  cid = lax.axis_index("c")
  sid = lax.axis_index("s")
  w = cid * NSUB + sid

  _zero_slices(sid, dacc, z128)
  pltpu.sync_copy(ones_h, onesv)
  plsc.subcore_barrier()

  def chunk(c, carry):
    pltpu.sync_copy(dstr2.at[w, pl.ds(c * CHK, CHK)], dstv)
    for k in range(CHK):
      pltpu.sync_copy(onesv, dacc.at[dstv.at[k]], add=True)
    return carry
  lax.fori_loop(0, DNCH, chunk, 0)

  plsc.subcore_barrier()

  @pl.when(cid == 0)
  def _():
    _out_slices(sid, dacc, d0)

  @pl.when(cid == 1)
  def _():
    _out_slices(sid, dacc, d1)


_deg = pl.kernel(
    _deg_body, mesh=_sc_mesh,
    out_type=[jax.ShapeDtypeStruct((ACC_ROWS, HALF), f32),
              jax.ShapeDtypeStruct((ACC_ROWS, HALF), f32)],
    scratch_types=[
        pltpu.VMEM((CHK, EB), jnp.int32),
        pltpu.VMEM((EB, HALF), f32),
        pltpu.VMEM_SHARED((ACC_ROWS, HALF), f32),
    ])


BN = 1000


def _full(shape):
  return pl.BlockSpec(shape, lambda i: (0, 0))


def _rows(w):
  return pl.BlockSpec((BN, w), lambda i: (i, 0))


def _encoder(x, W1, b1, W2, b2):
  def body(x_r, w1, b1r, w2, b2r, o0, o1):
    t = jnp.maximum(jnp.dot(x_r[...], w1[...], preferred_element_type=f32)
                    + b1r[...], 0.0)
    h = jnp.maximum(jnp.dot(t, w2[...], preferred_element_type=f32)
                    + b2r[...], 0.0)
    o0[...] = h[:, :HALF]
    o1[...] = h[:, HALF:]

  return pl.pallas_call(
      body,
      grid=(N // BN,),
      in_specs=[_rows(IN_D), _full((IN_D, HID)), _full((1, HID)),
                _full((HID, HID)), _full((1, HID))],
      out_specs=[_rows(HALF), _rows(HALF)],
      out_shape=[jax.ShapeDtypeStruct((N, HALF), f32),
                 jax.ShapeDtypeStruct((N, HALF), f32)],
  )(x, W1, b1.reshape(1, HID), W2, b2.reshape(1, HID))


def _self_term(h0, h1, Ws, b):
  def body(h0_r, h1_r, ws, br, s0, s1):
    h = jnp.concatenate([h0_r[...], h1_r[...]], axis=1)
    s = jnp.dot(h, ws[...], preferred_element_type=f32) + br[...]
    s0[...] = s[:, :HALF]
    s1[...] = s[:, HALF:]

  return pl.pallas_call(
      body,
      grid=(N // BN,),
      in_specs=[_rows(HALF), _rows(HALF), _full((HID, HID)), _full((1, HID))],
      out_specs=[_rows(HALF), _rows(HALF)],
      out_shape=[jax.ShapeDtypeStruct((N, HALF), f32),
                 jax.ShapeDtypeStruct((N, HALF), f32)],
  )(h0, h1, Ws, b.reshape(1, HID))


def _sage_post(s0, s1, a0, a1, d0, d1, Wn):
  def body(s0_r, s1_r, a0_r, a1_r, d0_r, d1_r, wn, o0, o1):
    s = jnp.concatenate([s0_r[...], s1_r[...]], axis=1)
    a = jnp.concatenate([a0_r[...], a1_r[...]], axis=1)
    a = a / jnp.maximum(d0_r[...] + d1_r[...], 1.0)
    o = jnp.maximum(s + jnp.dot(a, wn[...], preferred_element_type=f32), 0.0)
    o0[...] = o[:, :HALF]
    o1[...] = o[:, HALF:]

  return pl.pallas_call(
      body,
      grid=(N // BN,),
      in_specs=[_rows(HALF), _rows(HALF), _rows(HALF), _rows(HALF),
                pl.BlockSpec((BN, 1), lambda i: (i, 0)),
                pl.BlockSpec((BN, 1), lambda i: (i, 0)),
                _full((HID, HID))],
      out_specs=[_rows(HALF), _rows(HALF)],
      out_shape=[jax.ShapeDtypeStruct((N, HALF), f32),
                 jax.ShapeDtypeStruct((N, HALF), f32)],
  )(s0, s1, a0, a1, d0, d1, Wn)


def _classifier(h0, h1, Wc1, bc1, Wc2, bc2):
  def body(h0_r, h1_r, w1, b1r, w2, b2r, y_r, emb_r):
    h = jnp.concatenate([h0_r[...], h1_r[...]], axis=1)
    t = jnp.maximum(jnp.dot(h, w1[...], preferred_element_type=f32)
                    + b1r[...], 0.0)
    y_r[...] = jnp.dot(t, w2[...], preferred_element_type=f32) + b2r[...]
    emb_r[...] = h

  return pl.pallas_call(
      body,
      grid=(N // BN,),
      in_specs=[_rows(HALF), _rows(HALF), _full((HID, HID)), _full((1, HID)),
                _full((HID, OUT_D)), _full((1, OUT_D))],
      out_specs=[_rows(OUT_D), _rows(HID)],
      out_shape=[jax.ShapeDtypeStruct((N, OUT_D), f32),
                 jax.ShapeDtypeStruct((N, HID), f32)],
  )(h0, h1, Wc1, bc1.reshape(1, HID), Wc2, bc2.reshape(1, OUT_D))


def kernel(x, edge_index, W1, b1, W2, b2, Ws0, Wn0, bg0, Ws1, Wn1, bg1,
           Ws2, Wn2, bg2, Wc1, bc1, Wc2, bc2):
  pad = E_PAD - E
  src_p = jnp.concatenate([edge_index[0], jnp.zeros((pad,), jnp.int32)])
  dst_p = jnp.concatenate([edge_index[1], jnp.full((pad,), N, jnp.int32)])
  srcr = src_p.reshape(NSUB, BLOCKS, EB)
  dstr = dst_p.reshape(NSUB, BLOCKS, EB)
  dstr2 = dst_p.reshape(2 * NSUB, DBLOCKS, EB)
  z128 = jnp.zeros((ZROWS, HALF), f32)
  ones_h = jnp.ones((EB, HALF), f32)

  dd0, dd1 = _deg(dstr2, ones_h, z128)
  d0 = dd0[:, :1]
  d1 = dd1[:, :1]

  h0, h1 = _encoder(x, W1, b1, W2, b2)

  for (Ws, Wn, bg) in ((Ws0, Wn0, bg0), (Ws1, Wn1, bg1), (Ws2, Wn2, bg2)):
    a0, a1 = _agg(h0, h1, srcr, dstr, z128)
    s0, s1 = _self_term(h0, h1, Ws, bg)
    h0, h1 = _sage_post(s0, s1, a0, a1, d0, d1, Wn)

  return _classifier(h0, h1, Wc1, bc1, Wc2, bc2)


# speedup vs baseline: 1.1346x; 1.1346x over previous
import jax
import jax.numpy as jnp
from jax import lax
from jax.experimental import pallas as pl
from jax.experimental.pallas import tpu as pltpu
from jax.experimental.pallas import tpu_sc as plsc

f32 = jnp.float32

N = 10000
E = 320000
IN_D = 128
HID = 256
HALF = 128
OUT_D = 128

NSUB = 16
EB = 128
BLOCKS = 160
E_PAD = NSUB * BLOCKS * EB
CHK = 16
NCH = BLOCKS // CHK
DBLOCKS = 80
DNCH = DBLOCKS // CHK
ZROWS = 624
ACC_ROWS = 10008

_sc_mesh = plsc.VectorSubcoreMesh(core_axis_name="c", subcore_axis_name="s")


def _zero_slices(sid, dst_ref, z128):
  pltpu.sync_copy(z128, dst_ref.at[pl.ds(sid * ZROWS, ZROWS)])
  rem = ACC_ROWS - NSUB * ZROWS

  @pl.when(sid == 0)
  def _():
    pltpu.sync_copy(z128.at[pl.ds(0, rem)],
                    dst_ref.at[pl.ds(NSUB * ZROWS, rem)])


def _out_slices(sid, src_ref, out_ref):
  lo = sid * ZROWS
  pltpu.sync_copy(src_ref.at[pl.ds(lo, ZROWS)], out_ref.at[pl.ds(lo, ZROWS)])
  rem = ACC_ROWS - NSUB * ZROWS

  @pl.when(sid == 0)
  def _():
    pltpu.sync_copy(src_ref.at[pl.ds(NSUB * ZROWS, rem)],
                    out_ref.at[pl.ds(NSUB * ZROWS, rem)])


def _agg_body(h0, h1, srcr, dstr, z128, out0, out1, srcv, dstv, rows0, rows1,
              acc, sem, sem1):
  cid = lax.axis_index("c")
  sid = lax.axis_index("s")

  _zero_slices(sid, acc, z128)
  plsc.subcore_barrier()

  def run(h):
    def chunk(c, carry):
      pltpu.sync_copy(srcr.at[sid, pl.ds(c * CHK, CHK)], srcv)
      pltpu.sync_copy(dstr.at[sid, pl.ds(c * CHK, CHK)], dstv)
      pltpu.async_copy(h.at[srcv.at[0]], rows0, sem)
      pltpu.async_copy(h.at[srcv.at[1]], rows1, sem1)
      for k in range(CHK):
        cur = rows0 if k % 2 == 0 else rows1
        sm = sem if k % 2 == 0 else sem1
        pltpu.make_async_copy(h.at[srcv.at[k]], cur, sm).wait()
        pltpu.sync_copy(cur, acc.at[dstv.at[k]], add=True)
        if k + 2 < CHK:
          pltpu.async_copy(h.at[srcv.at[k + 2]], cur, sm)
      return carry
    lax.fori_loop(0, NCH, chunk, 0)

  @pl.when(cid == 0)
  def _():
    run(h0)

  @pl.when(cid == 1)
  def _():
    run(h1)

  plsc.subcore_barrier()

  @pl.when(cid == 0)
  def _():
    _out_slices(sid, acc, out0)

  @pl.when(cid == 1)
  def _():
    _out_slices(sid, acc, out1)


_agg = pl.kernel(
    _agg_body, mesh=_sc_mesh,
    out_type=[jax.ShapeDtypeStruct((ACC_ROWS, HALF), f32),
              jax.ShapeDtypeStruct((ACC_ROWS, HALF), f32)],
    scratch_types=[
        pltpu.VMEM((CHK, EB), jnp.int32),
        pltpu.VMEM((CHK, EB), jnp.int32),
        pltpu.VMEM((EB, HALF), f32),
        pltpu.VMEM((EB, HALF), f32),
        pltpu.VMEM_SHARED((ACC_ROWS, HALF), f32),
        pltpu.SemaphoreType.DMA,
        pltpu.SemaphoreType.DMA,
    ])


def _deg_body(dstr2, ones_h, z128, d0, d1, dstv, onesv, dacc):
  cid = lax.axis_index("c")
  sid = lax.axis_index("s")
  w = cid * NSUB + sid

  _zero_slices(sid, dacc, z128)
  pltpu.sync_copy(ones_h, onesv)
  plsc.subcore_barrier()

  def chunk(c, carry):
    pltpu.sync_copy(dstr2.at[w, pl.ds(c * CHK, CHK)], dstv)
    for k in range(CHK):
      pltpu.sync_copy(onesv, dacc.at[dstv.at[k]], add=True)
    return carry
  lax.fori_loop(0, DNCH, chunk, 0)

  plsc.subcore_barrier()

  @pl.when(cid == 0)
  def _():
    _out_slices(sid, dacc, d0)

  @pl.when(cid == 1)
  def _():
    _out_slices(sid, dacc, d1)


_deg = pl.kernel(
    _deg_body, mesh=_sc_mesh,
    out_type=[jax.ShapeDtypeStruct((ACC_ROWS, HALF), f32),
              jax.ShapeDtypeStruct((ACC_ROWS, HALF), f32)],
    scratch_types=[
        pltpu.VMEM((CHK, EB), jnp.int32),
        pltpu.VMEM((EB, HALF), f32),
        pltpu.VMEM_SHARED((ACC_ROWS, HALF), f32),
    ])


BN = 1000


def _full(shape):
  return pl.BlockSpec(shape, lambda i: (0, 0))


def _rows(w):
  return pl.BlockSpec((BN, w), lambda i: (i, 0))


def _encoder(x, W1, b1, W2, b2):
  def body(x_r, w1, b1r, w2, b2r, o0, o1):
    t = jnp.maximum(jnp.dot(x_r[...], w1[...], preferred_element_type=f32)
                    + b1r[...], 0.0)
    h = jnp.maximum(jnp.dot(t, w2[...], preferred_element_type=f32)
                    + b2r[...], 0.0)
    o0[...] = h[:, :HALF]
    o1[...] = h[:, HALF:]

  return pl.pallas_call(
      body,
      grid=(N // BN,),
      in_specs=[_rows(IN_D), _full((IN_D, HID)), _full((1, HID)),
                _full((HID, HID)), _full((1, HID))],
      out_specs=[_rows(HALF), _rows(HALF)],
      out_shape=[jax.ShapeDtypeStruct((N, HALF), f32),
                 jax.ShapeDtypeStruct((N, HALF), f32)],
  )(x, W1, b1.reshape(1, HID), W2, b2.reshape(1, HID))


def _sage_update(h0, h1, a0, a1, d0, d1, Ws, Wn, b):
  def body(h0_r, h1_r, a0_r, a1_r, d0_r, d1_r, ws, wn, br, o0, o1):
    h = jnp.concatenate([h0_r[...], h1_r[...]], axis=1)
    a = jnp.concatenate([a0_r[...], a1_r[...]], axis=1)
    a = a / jnp.maximum(d0_r[...] + d1_r[...], 1.0)
    o = jnp.maximum(jnp.dot(h, ws[...], preferred_element_type=f32)
                    + jnp.dot(a, wn[...], preferred_element_type=f32)
                    + br[...], 0.0)
    o0[...] = o[:, :HALF]
    o1[...] = o[:, HALF:]

  return pl.pallas_call(
      body,
      grid=(N // BN,),
      in_specs=[_rows(HALF), _rows(HALF), _rows(HALF), _rows(HALF),
                pl.BlockSpec((BN, 1), lambda i: (i, 0)),
                pl.BlockSpec((BN, 1), lambda i: (i, 0)),
                _full((HID, HID)), _full((HID, HID)), _full((1, HID))],
      out_specs=[_rows(HALF), _rows(HALF)],
      out_shape=[jax.ShapeDtypeStruct((N, HALF), f32),
                 jax.ShapeDtypeStruct((N, HALF), f32)],
  )(h0, h1, a0, a1, d0, d1, Ws, Wn, b.reshape(1, HID))


def _classifier(h0, h1, Wc1, bc1, Wc2, bc2):
  def body(h0_r, h1_r, w1, b1r, w2, b2r, y_r, emb_r):
    h = jnp.concatenate([h0_r[...], h1_r[...]], axis=1)
    t = jnp.maximum(jnp.dot(h, w1[...], preferred_element_type=f32)
                    + b1r[...], 0.0)
    y_r[...] = jnp.dot(t, w2[...], preferred_element_type=f32) + b2r[...]
    emb_r[...] = h

  return pl.pallas_call(
      body,
      grid=(N // BN,),
      in_specs=[_rows(HALF), _rows(HALF), _full((HID, HID)), _full((1, HID)),
                _full((HID, OUT_D)), _full((1, OUT_D))],
      out_specs=[_rows(OUT_D), _rows(HID)],
      out_shape=[jax.ShapeDtypeStruct((N, OUT_D), f32),
                 jax.ShapeDtypeStruct((N, HID), f32)],
  )(h0, h1, Wc1, bc1.reshape(1, HID), Wc2, bc2.reshape(1, OUT_D))


def kernel(x, edge_index, W1, b1, W2, b2, Ws0, Wn0, bg0, Ws1, Wn1, bg1,
           Ws2, Wn2, bg2, Wc1, bc1, Wc2, bc2):
  pad = E_PAD - E
  src_p = jnp.concatenate([edge_index[0], jnp.zeros((pad,), jnp.int32)])
  dst_p = jnp.concatenate([edge_index[1], jnp.full((pad,), N, jnp.int32)])
  srcr = src_p.reshape(NSUB, BLOCKS, EB)
  dstr = dst_p.reshape(NSUB, BLOCKS, EB)
  dstr2 = dst_p.reshape(2 * NSUB, DBLOCKS, EB)
  z128 = jnp.zeros((ZROWS, HALF), f32)
  ones_h = jnp.ones((EB, HALF), f32)

  dd0, dd1 = _deg(dstr2, ones_h, z128)
  d0 = dd0[:, :1]
  d1 = dd1[:, :1]

  h0, h1 = _encoder(x, W1, b1, W2, b2)

  a0, a1 = _agg(h0, h1, srcr, dstr, z128)
  h0, h1 = _sage_update(h0, h1, a0, a1, d0, d1, Ws0, Wn0, bg0)

  a0, a1 = _agg(h0, h1, srcr, dstr, z128)
  h0, h1 = _sage_update(h0, h1, a0, a1, d0, d1, Ws1, Wn1, bg1)

  a0, a1 = _agg(h0, h1, srcr, dstr, z128)
  h0, h1 = _sage_update(h0, h1, a0, a1, d0, d1, Ws2, Wn2, bg2)

  return _classifier(h0, h1, Wc1, bc1, Wc2, bc2)


# speedup vs baseline: 1.1414x; 1.0059x over previous
import jax
import jax.numpy as jnp
from jax import lax
from jax.experimental import pallas as pl
from jax.experimental.pallas import tpu as pltpu
from jax.experimental.pallas import tpu_sc as plsc

f32 = jnp.float32

N = 10000
E = 320000
IN_D = 128
HID = 256
HALF = 128
OUT_D = 128

NSUB = 16
EB = 128
BLOCKS = 160
E_PAD = NSUB * BLOCKS * EB
CHK = 8
NCH = BLOCKS // CHK
NPAIR = BLOCKS // (2 * CHK)
DBLOCKS = 80
DNCH = DBLOCKS // CHK
ZROWS = 624
ACC_ROWS = 10008

_sc_mesh = plsc.VectorSubcoreMesh(core_axis_name="c", subcore_axis_name="s")


def _zero_slices(sid, dst_ref, z128):
  pltpu.sync_copy(z128, dst_ref.at[pl.ds(sid * ZROWS, ZROWS)])
  rem = ACC_ROWS - NSUB * ZROWS

  @pl.when(sid == 0)
  def _():
    pltpu.sync_copy(z128.at[pl.ds(0, rem)],
                    dst_ref.at[pl.ds(NSUB * ZROWS, rem)])


def _out_slices(sid, src_ref, out_ref):
  lo = sid * ZROWS
  pltpu.sync_copy(src_ref.at[pl.ds(lo, ZROWS)], out_ref.at[pl.ds(lo, ZROWS)])
  rem = ACC_ROWS - NSUB * ZROWS

  @pl.when(sid == 0)
  def _():
    pltpu.sync_copy(src_ref.at[pl.ds(NSUB * ZROWS, rem)],
                    out_ref.at[pl.ds(NSUB * ZROWS, rem)])


def _agg_body(h0, h1, srcr, dstr, z128, out0, out1, srcA, srcB, dstA, dstB,
              rows0, rows1, acc, sem, sem1):
  cid = lax.axis_index("c")
  sid = lax.axis_index("s")

  _zero_slices(sid, acc, z128)
  plsc.subcore_barrier()

  def stage(c, sv, dv):
    pltpu.sync_copy(srcr.at[sid, pl.ds(c * CHK, CHK)], sv)
    pltpu.sync_copy(dstr.at[sid, pl.ds(c * CHK, CHK)], dv)

  def run(h):
    stage(0, srcA, dstA)
    pltpu.async_copy(h.at[srcA.at[0]], rows0, sem)
    pltpu.async_copy(h.at[srcA.at[1]], rows1, sem1)

    def pair(i, carry):
      stage(2 * i + 1, srcB, dstB)
      for (sv, dv, other, last) in ((srcA, dstA, srcB, False),
                                    (srcB, dstB, srcA, True)):
        if last:
          @pl.when(i < NPAIR - 1)
          def _():
            stage(2 * i + 2, srcA, dstA)
        for k in range(CHK):
          cur = rows0 if k % 2 == 0 else rows1
          sm = sem if k % 2 == 0 else sem1
          pltpu.make_async_copy(h.at[sv.at[k]], cur, sm).wait()
          pltpu.sync_copy(cur, acc.at[dv.at[k]], add=True)
          if k + 2 < CHK:
            pltpu.async_copy(h.at[sv.at[k + 2]], cur, sm)
          elif not last:
            pltpu.async_copy(h.at[other.at[k + 2 - CHK]], cur, sm)
          else:
            @pl.when(i < NPAIR - 1)
            def _():
              pltpu.async_copy(h.at[other.at[k + 2 - CHK]], cur, sm)
      return carry
    lax.fori_loop(0, NPAIR, pair, 0)

  @pl.when(cid == 0)
  def _():
    run(h0)

  @pl.when(cid == 1)
  def _():
    run(h1)

  plsc.subcore_barrier()

  @pl.when(cid == 0)
  def _():
    _out_slices(sid, acc, out0)

  @pl.when(cid == 1)
  def _():
    _out_slices(sid, acc, out1)


_agg = pl.kernel(
    _agg_body, mesh=_sc_mesh,
    out_type=[jax.ShapeDtypeStruct((ACC_ROWS, HALF), f32),
              jax.ShapeDtypeStruct((ACC_ROWS, HALF), f32)],
    scratch_types=[
        pltpu.VMEM((CHK, EB), jnp.int32),
        pltpu.VMEM((CHK, EB), jnp.int32),
        pltpu.VMEM((CHK, EB), jnp.int32),
        pltpu.VMEM((CHK, EB), jnp.int32),
        pltpu.VMEM((EB, HALF), f32),
        pltpu.VMEM((EB, HALF), f32),
        pltpu.VMEM_SHARED((ACC_ROWS, HALF), f32),
        pltpu.SemaphoreType.DMA,
        pltpu.SemaphoreType.DMA,
    ])


def _deg_body(dstr2, ones_h, z128, d0, d1, dstv, onesv, dacc):
  cid = lax.axis_index("c")
  sid = lax.axis_index("s")
  w = cid * NSUB + sid

  _zero_slices(sid, dacc, z128)
  pltpu.sync_copy(ones_h, onesv)
  plsc.subcore_barrier()

  def chunk(c, carry):
    pltpu.sync_copy(dstr2.at[w, pl.ds(c * CHK, CHK)], dstv)
    for k in range(CHK):
      pltpu.sync_copy(onesv, dacc.at[dstv.at[k]], add=True)
    return carry
  lax.fori_loop(0, DNCH, chunk, 0)

  plsc.subcore_barrier()

  @pl.when(cid == 0)
  def _():
    _out_slices(sid, dacc, d0)

  @pl.when(cid == 1)
  def _():
    _out_slices(sid, dacc, d1)


_deg = pl.kernel(
    _deg_body, mesh=_sc_mesh,
    out_type=[jax.ShapeDtypeStruct((ACC_ROWS, HALF), f32),
              jax.ShapeDtypeStruct((ACC_ROWS, HALF), f32)],
    scratch_types=[
        pltpu.VMEM((CHK, EB), jnp.int32),
        pltpu.VMEM((EB, HALF), f32),
        pltpu.VMEM_SHARED((ACC_ROWS, HALF), f32),
    ])


BN = 1000


def _full(shape):
  return pl.BlockSpec(shape, lambda i: (0, 0))


def _rows(w):
  return pl.BlockSpec((BN, w), lambda i: (i, 0))


def _encoder(x, W1, b1, W2, b2):
  def body(x_r, w1, b1r, w2, b2r, o0, o1):
    t = jnp.maximum(jnp.dot(x_r[...], w1[...], preferred_element_type=f32)
                    + b1r[...], 0.0)
    h = jnp.maximum(jnp.dot(t, w2[...], preferred_element_type=f32)
                    + b2r[...], 0.0)
    o0[...] = h[:, :HALF]
    o1[...] = h[:, HALF:]

  return pl.pallas_call(
      body,
      grid=(N // BN,),
      in_specs=[_rows(IN_D), _full((IN_D, HID)), _full((1, HID)),
                _full((HID, HID)), _full((1, HID))],
      out_specs=[_rows(HALF), _rows(HALF)],
      out_shape=[jax.ShapeDtypeStruct((N, HALF), f32),
                 jax.ShapeDtypeStruct((N, HALF), f32)],
  )(x, W1, b1.reshape(1, HID), W2, b2.reshape(1, HID))


def _sage_update(h0, h1, a0, a1, d0, d1, Ws, Wn, b):
  def body(h0_r, h1_r, a0_r, a1_r, d0_r, d1_r, ws, wn, br, o0, o1):
    h = jnp.concatenate([h0_r[...], h1_r[...]], axis=1)
    a = jnp.concatenate([a0_r[...], a1_r[...]], axis=1)
    a = a / jnp.maximum(d0_r[...] + d1_r[...], 1.0)
    o = jnp.maximum(jnp.dot(h, ws[...], preferred_element_type=f32)
                    + jnp.dot(a, wn[...], preferred_element_type=f32)
                    + br[...], 0.0)
    o0[...] = o[:, :HALF]
    o1[...] = o[:, HALF:]

  return pl.pallas_call(
      body,
      grid=(N // BN,),
      in_specs=[_rows(HALF), _rows(HALF), _rows(HALF), _rows(HALF),
                pl.BlockSpec((BN, 1), lambda i: (i, 0)),
                pl.BlockSpec((BN, 1), lambda i: (i, 0)),
                _full((HID, HID)), _full((HID, HID)), _full((1, HID))],
      out_specs=[_rows(HALF), _rows(HALF)],
      out_shape=[jax.ShapeDtypeStruct((N, HALF), f32),
                 jax.ShapeDtypeStruct((N, HALF), f32)],
  )(h0, h1, a0, a1, d0, d1, Ws, Wn, b.reshape(1, HID))


def _classifier(h0, h1, Wc1, bc1, Wc2, bc2):
  def body(h0_r, h1_r, w1, b1r, w2, b2r, y_r, emb_r):
    h = jnp.concatenate([h0_r[...], h1_r[...]], axis=1)
    t = jnp.maximum(jnp.dot(h, w1[...], preferred_element_type=f32)
                    + b1r[...], 0.0)
    y_r[...] = jnp.dot(t, w2[...], preferred_element_type=f32) + b2r[...]
    emb_r[...] = h

  return pl.pallas_call(
      body,
      grid=(N // BN,),
      in_specs=[_rows(HALF), _rows(HALF), _full((HID, HID)), _full((1, HID)),
                _full((HID, OUT_D)), _full((1, OUT_D))],
      out_specs=[_rows(OUT_D), _rows(HID)],
      out_shape=[jax.ShapeDtypeStruct((N, OUT_D), f32),
                 jax.ShapeDtypeStruct((N, HID), f32)],
  )(h0, h1, Wc1, bc1.reshape(1, HID), Wc2, bc2.reshape(1, OUT_D))


def kernel(x, edge_index, W1, b1, W2, b2, Ws0, Wn0, bg0, Ws1, Wn1, bg1,
           Ws2, Wn2, bg2, Wc1, bc1, Wc2, bc2):
  pad = E_PAD - E
  src_p = jnp.concatenate([edge_index[0], jnp.zeros((pad,), jnp.int32)])
  dst_p = jnp.concatenate([edge_index[1], jnp.full((pad,), N, jnp.int32)])
  srcr = src_p.reshape(NSUB, BLOCKS, EB)
  dstr = dst_p.reshape(NSUB, BLOCKS, EB)
  dstr2 = dst_p.reshape(2 * NSUB, DBLOCKS, EB)
  z128 = jnp.zeros((ZROWS, HALF), f32)
  ones_h = jnp.ones((EB, HALF), f32)

  dd0, dd1 = _deg(dstr2, ones_h, z128)
  d0 = dd0[:, :1]
  d1 = dd1[:, :1]

  h0, h1 = _encoder(x, W1, b1, W2, b2)

  a0, a1 = _agg(h0, h1, srcr, dstr, z128)
  h0, h1 = _sage_update(h0, h1, a0, a1, d0, d1, Ws0, Wn0, bg0)

  a0, a1 = _agg(h0, h1, srcr, dstr, z128)
  h0, h1 = _sage_update(h0, h1, a0, a1, d0, d1, Ws1, Wn1, bg1)

  a0, a1 = _agg(h0, h1, srcr, dstr, z128)
  h0, h1 = _sage_update(h0, h1, a0, a1, d0, d1, Ws2, Wn2, bg2)

  return _classifier(h0, h1, Wc1, bc1, Wc2, bc2)
